# baseline (speedup 1.0000x reference)
import functools

import jax
import jax.numpy as jnp
from jax import lax
from jax.experimental import pallas as pl
from jax.experimental.pallas import tpu as pltpu
from jax.experimental.pallas import tpu_sc as plsc

_B = 256
_H = 128
_W = 128
_P = 32
_NW = 32
_BPW = _B // _NW
_CH = 16
_NCH = _H // _CH
_CHW = _CH * _P * _W
_IMG = _H * _P * _W


def _sc_body(xc_hbm, yc_hbm, scal_hbm, zin_hbm, out_hbm, xv, yv, sv, zbuf,
             idxv, onev, fsem, ssem):
    wid = lax.axis_index("s") * 2 + lax.axis_index("c")
    b0 = wid * _BPW
    pltpu.sync_copy(zin_hbm, zbuf)
    pltpu.sync_copy(xc_hbm.at[pl.ds(b0, _BPW)], xv)
    pltpu.sync_copy(yc_hbm.at[pl.ds(b0, _BPW)], yv)
    pltpu.sync_copy(scal_hbm.at[pl.ds(b0, _BPW)], sv)
    iota = lax.broadcasted_iota(jnp.int32, (16,), 0)
    ones = jnp.full((16,), 1.0, jnp.float32)
    for i in range(8):
        onev[pl.ds(i * 16, 16)] = ones
    prev = None
    for bi in range(_BPW):
        b = b0 + bi
        rx = sv[bi, 0]
        ry = sv[bi, 1]
        ox = sv[bi, 2]
        oy = sv[bi, 3]
        for g in range(2):
            col = (xv[bi, g] / rx + ox).astype(jnp.int32)
            row = (yv[bi, g] / ry + oy).astype(jnp.int32)
            ip = iota + 16 * g
            fi = b * _IMG + row * (_P * _W) + ip * _W + col
            k = bi * 2 + g
            idxv[k // 8, pl.ds((k % 8) * 16, 16)] = fi
        handles = []
        for c in range(_NCH):
            handles.append(
                pltpu.async_copy(
                    zbuf, out_hbm.at[pl.ds(b * _IMG + c * _CHW, _CHW)], fsem
                )
            )
        if prev is not None:
            for h in prev:
                h.wait()
        prev = handles
    for h in prev:
        h.wait()
    s0 = pltpu.async_copy(onev, out_hbm.at[idxv.at[0]], ssem)
    s1 = pltpu.async_copy(onev, out_hbm.at[idxv.at[1]], ssem)
    s0.wait()
    s1.wait()


def kernel(x, resolution, origin):
    B = x.shape[0]
    pts = x.reshape(B, _P, 2)
    xc = pts[:, :, 0].reshape(B, 2, 16)
    yc = pts[:, :, 1].reshape(B, 2, 16)
    scal = jnp.stack(
        [
            jnp.broadcast_to(resolution[:, 0:1], (B, 16)),
            jnp.broadcast_to(resolution[:, 1:2], (B, 16)),
            jnp.broadcast_to(origin[:, 0:1], (B, 16)),
            jnp.broadcast_to(origin[:, 1:2], (B, 16)),
        ],
        axis=1,
    )
    zin = jnp.zeros((_CHW,), jnp.float32)
    run = functools.partial(
        pl.kernel,
        out_type=jax.ShapeDtypeStruct((B * _IMG,), jnp.float32),
        mesh=plsc.VectorSubcoreMesh(core_axis_name="c", subcore_axis_name="s"),
        compiler_params=pltpu.CompilerParams(
            needs_layout_passes=False, use_tc_tiling_on_sc=False
        ),
        scratch_types=[
            pltpu.VMEM((_BPW, 2, 16), jnp.float32),
            pltpu.VMEM((_BPW, 2, 16), jnp.float32),
            pltpu.VMEM((_BPW, 4, 16), jnp.float32),
            pltpu.VMEM((_CHW,), jnp.float32),
            pltpu.VMEM((2, 128), jnp.int32),
            pltpu.VMEM((128,), jnp.float32),
            pltpu.SemaphoreType.DMA,
            pltpu.SemaphoreType.DMA,
        ],
    )(_sc_body)
    out = run(xc, yc, scal, zin)
    return jnp.transpose(out.reshape(B, _H, _P, _W), (0, 1, 3, 2))
